# Initial kernel scaffold; baseline (speedup 1.0000x reference)
import jax
import jax.numpy as jnp
from jax.experimental import pallas as pl


def kernel(input_len, device, table):
    raise NotImplementedError("write your pallas kernel here")


# speedup vs baseline: 1.0829x; 1.0829x over previous
import numpy as np
import jax
import jax.numpy as jnp
from jax import lax
from jax.experimental import pallas as pl
from jax.experimental.pallas import tpu as pltpu
from jax.experimental.pallas import tpu_sc as plsc

D_MODEL = 64
MAX_SEQ_LEN = 200
BATCH = 4096

_G = 8
_NGRP = MAX_SEQ_LEN // _G
_GW = _G * D_MODEL
_ZROW = MAX_SEQ_LEN

_NC = 2
_NS = 16
_NW = _NC * _NS
_ROWS_PER_W = BATCH // _NW
_ENT_PER_W = _ROWS_PER_W * _NGRP
_CHUNK = 64
_NCHUNKS = _ENT_PER_W // _CHUNK
_POS_PER_CHUNK = _CHUNK * _G
_L = 16

_IDS = np.zeros((_NGRP, _G, _G), dtype=np.int32)
for _t in range(_NGRP):
    for _c in range(1, _G + 1):
        for _j in range(_G):
            _IDS[_t, _c - 1, _j] = (8 * _t + _j + 1) if _j < _c else 0
_GROUP_IDS = _IDS.reshape(-1)


def _pe_body(len_hbm, gtab_hbm, emb_hbm, pos_hbm,
             len_v, gidx_v, pid_v, rows_v, sem):
    wid = lax.axis_index("s") * _NC + lax.axis_index("c")
    row_base = wid * _ROWS_PER_W
    ent_base = wid * _ENT_PER_W
    pos_base = wid * _ROWS_PER_W * MAX_SEQ_LEN
    pltpu.sync_copy(len_hbm.at[pl.ds(row_base, _ROWS_PER_W)], len_v)

    def chunk(j, carry):
        e0 = j * _CHUNK
        for v in range(_CHUNK // _L):
            e = e0 + v * _L + lax.iota(jnp.int32, _L)
            r = e // _NGRP
            t = e - r * _NGRP
            lenr = plsc.load_gather(len_v, [r])
            kept = jnp.clip(lenr - _G * t, 0, _G)
            gidx_v[pl.ds(v * _L, _L)] = jnp.where(
                kept >= 1, _G * t + kept - 1, _ZROW)
        p0 = e0 * _G
        for v in range(_POS_PER_CHUNK // _L):
            p = p0 + v * _L + lax.iota(jnp.int32, _L)
            r = p // MAX_SEQ_LEN
            pos = p - r * MAX_SEQ_LEN + 1
            lenr = plsc.load_gather(len_v, [r])
            pid_v[pl.ds(v * _L, _L)] = jnp.where(pos <= lenr, pos, 0)
        pltpu.async_copy(gtab_hbm.at[gidx_v], rows_v, sem).wait()
        pltpu.sync_copy(rows_v, emb_hbm.at[pl.ds(ent_base + e0, _CHUNK)])
        pltpu.sync_copy(pid_v, pos_hbm.at[pl.ds(pos_base + p0,
                                                _POS_PER_CHUNK)])
        return carry

    lax.fori_loop(0, _NCHUNKS, chunk, 0)


def kernel(input_len, device, table):
    del device
    grouped = jnp.concatenate(
        [jnp.take(table, _GROUP_IDS, axis=0).reshape(MAX_SEQ_LEN, _GW),
         jnp.zeros((1, _GW), jnp.float32)], axis=0)
    mesh = plsc.VectorSubcoreMesh(core_axis_name="c", subcore_axis_name="s")
    k = pl.kernel(
        _pe_body,
        mesh=mesh,
        compiler_params=pltpu.CompilerParams(needs_layout_passes=False),
        out_type=[
            jax.ShapeDtypeStruct((BATCH * _NGRP, _GW), jnp.float32),
            jax.ShapeDtypeStruct((BATCH * MAX_SEQ_LEN,), jnp.int32),
        ],
        scratch_types=[
            pltpu.VMEM((_ROWS_PER_W,), jnp.int32),
            pltpu.VMEM((_CHUNK,), jnp.int32),
            pltpu.VMEM((_POS_PER_CHUNK,), jnp.int32),
            pltpu.VMEM((_CHUNK, _GW), jnp.float32),
            pltpu.SemaphoreType.DMA,
        ],
    )
    emb_flat, pos_flat = k(input_len.astype(jnp.int32), grouped)
    return (emb_flat.reshape(BATCH, MAX_SEQ_LEN, D_MODEL),
            pos_flat.reshape(BATCH, MAX_SEQ_LEN))
